# Initial kernel scaffold; baseline (speedup 1.0000x reference)
import jax
import jax.numpy as jnp
from jax.experimental import pallas as pl


def kernel(x, gate_w, expert_w):
    raise NotImplementedError("write your pallas kernel here")


# speedup vs baseline: 2.2636x; 2.2636x over previous
import functools

import jax
import jax.numpy as jnp
from jax.experimental import pallas as pl

NUM_EXPERTS = 8
TOP_K = 2
HIDDEN = 1024
NUM_TOKENS = 4096

BT = 512


def _moe_block(x_ref, gate_ref, ew_ref, out_ref):
    x = x_ref[...]
    gate_w = gate_ref[...]
    logits = jax.lax.dot_general(
        x, gate_w, (((1,), (1,)), ((), ())),
        precision=jax.lax.Precision.DEFAULT,
        preferred_element_type=jnp.float32)
    m = jnp.max(logits, axis=-1, keepdims=True)
    el = jnp.exp(logits - m)
    probs = el / jnp.sum(el, axis=-1, keepdims=True)

    eidx = jax.lax.broadcasted_iota(jnp.int32, probs.shape, 1)
    m1 = jnp.max(probs, axis=-1, keepdims=True)
    i1 = jnp.min(jnp.where(probs == m1, eidx, NUM_EXPERTS), axis=-1,
                 keepdims=True)
    oh1 = (eidx == i1).astype(jnp.float32)
    probs2 = jnp.where(oh1 > 0, -1.0, probs)
    m2 = jnp.max(probs2, axis=-1, keepdims=True)
    i2 = jnp.min(jnp.where(probs2 == m2, eidx, NUM_EXPERTS), axis=-1,
                 keepdims=True)
    oh2 = (eidx == i2).astype(jnp.float32)
    wexp = m1 * oh1 + m2 * oh2

    acc = jnp.zeros((x.shape[0], HIDDEN), dtype=jnp.float32)
    for e in range(NUM_EXPERTS):
        y = jax.lax.dot_general(
            x, ew_ref[e], (((1,), (1,)), ((), ())),
            precision=jax.lax.Precision.DEFAULT,
            preferred_element_type=jnp.float32)
        acc = acc + wexp[:, e:e + 1] * y
    out_ref[...] = acc


@jax.jit
def kernel(x, gate_w, expert_w):
    grid = (NUM_TOKENS // BT,)
    return pl.pallas_call(
        _moe_block,
        grid=grid,
        in_specs=[
            pl.BlockSpec((BT, HIDDEN), lambda i: (i, 0)),
            pl.BlockSpec((NUM_EXPERTS, HIDDEN), lambda i: (0, 0)),
            pl.BlockSpec((NUM_EXPERTS, HIDDEN, HIDDEN), lambda i: (0, 0, 0)),
        ],
        out_specs=pl.BlockSpec((BT, HIDDEN), lambda i: (i, 0)),
        out_shape=jax.ShapeDtypeStruct((NUM_TOKENS, HIDDEN), jnp.float32),
    )(x, gate_w, expert_w)


# speedup vs baseline: 2.2682x; 1.0020x over previous
import functools

import jax
import jax.numpy as jnp
from jax.experimental import pallas as pl

NUM_EXPERTS = 8
TOP_K = 2
HIDDEN = 1024
NUM_TOKENS = 4096

BT = 512


def _moe_block(x_ref, gate_ref, ew_ref, out_ref):
    x = x_ref[...]
    gate_w = gate_ref[...]
    logits = jax.lax.dot_general(
        x, gate_w, (((1,), (1,)), ((), ())),
        precision=jax.lax.Precision.DEFAULT,
        preferred_element_type=jnp.float32)
    m = jnp.max(logits, axis=-1, keepdims=True)
    el = jnp.exp(logits - m)
    probs = el / jnp.sum(el, axis=-1, keepdims=True)

    eidx = jax.lax.broadcasted_iota(jnp.int32, probs.shape, 1)
    m1 = jnp.max(probs, axis=-1, keepdims=True)
    i1 = jnp.min(jnp.where(probs == m1, eidx, NUM_EXPERTS), axis=-1,
                 keepdims=True)
    oh1 = (eidx == i1).astype(jnp.float32)
    probs2 = jnp.where(oh1 > 0, -1.0, probs)
    m2 = jnp.max(probs2, axis=-1, keepdims=True)
    i2 = jnp.min(jnp.where(probs2 == m2, eidx, NUM_EXPERTS), axis=-1,
                 keepdims=True)
    oh2 = (eidx == i2).astype(jnp.float32)
    wexp = m1 * oh1 + m2 * oh2

    acc = jnp.zeros((x.shape[0], HIDDEN), dtype=jnp.float32)
    xb = x.astype(jnp.bfloat16)
    for e in range(NUM_EXPERTS):
        y = jax.lax.dot_general(
            xb, ew_ref[e].astype(jnp.bfloat16), (((1,), (1,)), ((), ())),
            precision=jax.lax.Precision.DEFAULT,
            preferred_element_type=jnp.float32)
        acc = acc + wexp[:, e:e + 1] * y
    out_ref[...] = acc


@jax.jit
def kernel(x, gate_w, expert_w):
    grid = (NUM_TOKENS // BT,)
    return pl.pallas_call(
        _moe_block,
        grid=grid,
        in_specs=[
            pl.BlockSpec((BT, HIDDEN), lambda i: (i, 0)),
            pl.BlockSpec((NUM_EXPERTS, HIDDEN), lambda i: (0, 0)),
            pl.BlockSpec((NUM_EXPERTS, HIDDEN, HIDDEN), lambda i: (0, 0, 0)),
        ],
        out_specs=pl.BlockSpec((BT, HIDDEN), lambda i: (i, 0)),
        out_shape=jax.ShapeDtypeStruct((NUM_TOKENS, HIDDEN), jnp.float32),
    )(x, gate_w, expert_w)
